# baseline (speedup 1.0000x reference)
import functools

import jax
import jax.numpy as jnp
from jax import lax
from jax.experimental import pallas as pl
from jax.experimental.pallas import tpu as pltpu
from jax.experimental.pallas import tpu_sc as plsc

HIDDEN = 768
FF = 3072
E = 8
TOKENS = 2048

T = 256
G_MAX = 24
P_MAX = G_MAX * T
PT = 512
FF_BLK = 512
NC = 2
NW = 32
ROWS_W = P_MAX // NW
GCH = 64
TOK_W = TOKENS // NW


def _route_body(h_ref, wg_ref, tw_ref, d01_ref, eotv_ref):
    h = h_ref[...]
    logits = jnp.dot(h, wg_ref[...], preferred_element_type=jnp.float32)
    ids = lax.broadcasted_iota(jnp.int32, logits.shape, 1)
    m0 = jnp.max(logits, axis=-1, keepdims=True)
    a0 = jnp.min(jnp.where(logits == m0, ids, E), axis=-1, keepdims=True)
    l2 = jnp.where(ids == a0, -jnp.inf, logits)
    m1 = jnp.max(l2, axis=-1, keepdims=True)
    a1 = jnp.min(jnp.where(l2 == m1, ids, E), axis=-1, keepdims=True)
    p1 = jnp.exp(m1 - m0)
    denom = 1.0 + p1
    w0 = 1.0 / denom
    w1 = p1 / denom

    oh0 = ids == a0
    oh1 = ids == a1
    s = oh0.astype(jnp.float32) + oh1.astype(jnp.float32)
    incl = s
    k = 1
    while k < TOKENS:
        shifted = jnp.concatenate(
            [jnp.zeros((k, E), jnp.float32), incl[: TOKENS - k, :]], axis=0)
        incl = incl + shifted
        k *= 2
    base = incl - s
    rank0 = jnp.sum(jnp.where(oh0, base, 0.0), axis=1, keepdims=True)
    rank1 = jnp.sum(jnp.where(oh1, base, 0.0), axis=1, keepdims=True)

    totals = incl[TOKENS - 1:TOKENS, :]
    totals_i = totals.astype(jnp.int32)
    tiles_i = (totals_i + (T - 1)) // T
    r8 = lax.broadcasted_iota(jnp.int32, (E, E), 0)
    c8 = lax.broadcasted_iota(jnp.int32, (E, E), 1)
    lt = (r8 <= c8).astype(jnp.float32)
    cumt = jnp.dot(tiles_i.astype(jnp.float32), lt,
                   preferred_element_type=jnp.float32)
    pbase = (cumt - tiles_i.astype(jnp.float32)) * float(T)

    dest0 = jnp.sum(jnp.where(oh0, pbase, 0.0), axis=1, keepdims=True) + rank0
    dest1 = jnp.sum(jnp.where(oh1, pbase, 0.0), axis=1, keepdims=True) + rank1
    dest0_i = dest0.astype(jnp.int32)
    dest1_i = dest1.astype(jnp.int32)
    d01_ref[...] = (jnp.where(ids == 0, dest0_i, 0)
                    + jnp.where(ids == 1, dest1_i, 0))

    cumt_i = cumt.astype(jnp.int32)
    g_col = lax.broadcasted_iota(jnp.int32, (G_MAX, 1), 0)
    eot_full = jnp.sum((cumt_i <= g_col).astype(jnp.int32),
                       axis=1, keepdims=True)
    total_tiles = cumt_i[:, E - 1:E]
    valid = (g_col < total_tiles).astype(jnp.int32)
    eotc = jnp.minimum(eot_full, E - 1)
    lane_g = lax.broadcasted_iota(jnp.int32, (G_MAX, E), 1)
    eotv_ref[...] = (jnp.where(lane_g == 0, eotc, 0)
                     + jnp.where(lane_g == 1, valid, 0))

    tid_f = lax.broadcasted_iota(jnp.int32, (TOKENS, 1), 0).astype(jnp.float32)
    v0 = jnp.where(ids == 0, tid_f, 0.0) + jnp.where(ids == 1, w0, 0.0)
    v1 = jnp.where(ids == 0, tid_f, 0.0) + jnp.where(ids == 1, w1, 0.0)
    dn = (((0,), (0,)), ((), ()))
    for i in range(P_MAX // PT):
        q = lax.broadcasted_iota(jnp.int32, (TOKENS, PT), 1) + i * PT
        m0f = (q == dest0_i).astype(jnp.float32)
        m1f = (q == dest1_i).astype(jnp.float32)
        tw_ref[i * PT:(i + 1) * PT, :] = (
            lax.dot_general(m0f, v0, dn, precision=lax.Precision.HIGHEST,
                            preferred_element_type=jnp.float32)
            + lax.dot_general(m1f, v1, dn, precision=lax.Precision.HIGHEST,
                              preferred_element_type=jnp.float32))


def _ffn_body(eot_ref, valid_ref, xg_ref, w1_ref, w2_ref, tw_ref, y_ref):
    f = pl.program_id(0)
    g = pl.program_id(1)
    rows = pl.ds(g * T, T)

    @pl.when(valid_ref[g] == 1)
    def _():
        x = xg_ref[rows, :].astype(jnp.bfloat16)
        w1 = w1_ref[0].astype(jnp.bfloat16)
        w2 = w2_ref[0].astype(jnp.bfloat16)
        pre = jnp.dot(x, w1, preferred_element_type=jnp.float32)
        act = (pre * jax.nn.sigmoid(pre)).astype(jnp.bfloat16)
        contrib = jnp.dot(act, w2, preferred_element_type=jnp.float32)

        @pl.when(f == 0)
        def _():
            y_ref[rows, :] = contrib

        @pl.when(jnp.logical_and(f > 0, f < FF // FF_BLK - 1))
        def _():
            y_ref[rows, :] += contrib

        @pl.when(f == FF // FF_BLK - 1)
        def _():
            tw = tw_ref[rows, :]
            lane = lax.broadcasted_iota(jnp.int32, tw.shape, 1)
            wt = jnp.sum(jnp.where(lane == 1, tw, 0.0), axis=1, keepdims=True)
            y_ref[rows, :] = (y_ref[rows, :] + contrib) * wt


@functools.lru_cache(maxsize=None)
def _make_sc_gather():
    mesh = plsc.VectorSubcoreMesh(core_axis_name="c", subcore_axis_name="s")

    @functools.partial(
        pl.kernel,
        out_type=jax.ShapeDtypeStruct((P_MAX, HIDDEN), jnp.float32),
        mesh=mesh,
        scratch_types=[
            pltpu.VMEM((ROWS_W // GCH, GCH), jnp.int32),
            pltpu.VMEM((GCH, HIDDEN), jnp.float32),
            pltpu.VMEM((GCH, HIDDEN), jnp.float32),
            pltpu.SemaphoreType.DMA,
            pltpu.SemaphoreType.DMA,
        ],
    )
    def sc_gather(idx_hbm, h_hbm, out_hbm, idx_v, r0, r1, s0, s1):
        wid = lax.axis_index("s") * NC + lax.axis_index("c")
        base = wid * ROWS_W
        nch = ROWS_W // GCH
        for c in range(nch):
            pltpu.sync_copy(idx_hbm.at[pl.ds(base + c * GCH, GCH)],
                            idx_v.at[c])
        bufs = (r0, r1)
        sems = (s0, s1)
        cps = [None] * nch
        cps[0] = pltpu.async_copy(h_hbm.at[idx_v.at[0]], bufs[0], sems[0])
        for c in range(nch):
            if c + 1 < nch:
                cps[c + 1] = pltpu.async_copy(
                    h_hbm.at[idx_v.at[c + 1]], bufs[(c + 1) % 2],
                    sems[(c + 1) % 2])
            cps[c].wait()
            pltpu.sync_copy(bufs[c % 2],
                            out_hbm.at[pl.ds(base + c * GCH, GCH)])

    return sc_gather


def _sc_gather(sidx, h):
    return _make_sc_gather()(sidx, h)


@functools.lru_cache(maxsize=None)
def _make_sc_combine():
    mesh = plsc.VectorSubcoreMesh(core_axis_name="c", subcore_axis_name="s")

    @functools.partial(
        pl.kernel,
        out_type=jax.ShapeDtypeStruct((TOKENS, HIDDEN), jnp.float32),
        mesh=mesh,
        scratch_types=[
            pltpu.VMEM((TOK_W,), jnp.int32),
            pltpu.VMEM((TOK_W,), jnp.int32),
            pltpu.VMEM((TOK_W, HIDDEN), jnp.float32),
            pltpu.VMEM((TOK_W, HIDDEN), jnp.float32),
            pltpu.SemaphoreType.DMA,
            pltpu.SemaphoreType.DMA,
        ],
    )
    def sc_combine(pos0_hbm, pos1_hbm, rows_hbm, y_hbm,
                   i0, i1, r0, r1, s0, s1):
        wid = lax.axis_index("s") * NC + lax.axis_index("c")
        base = wid * TOK_W
        pltpu.sync_copy(pos0_hbm.at[pl.ds(base, TOK_W)], i0)
        pltpu.sync_copy(pos1_hbm.at[pl.ds(base, TOK_W)], i1)
        cp0 = pltpu.async_copy(rows_hbm.at[i0], r0, s0)
        cp1 = pltpu.async_copy(rows_hbm.at[i1], r1, s1)
        cp0.wait()
        cp1.wait()

        def add_row(t, carry):
            for j in range(HIDDEN // 16):
                sl = pl.ds(j * 16, 16)
                r0[t, sl] = r0[t, sl] + r1[t, sl]
            return carry

        lax.fori_loop(0, TOK_W, add_row, 0)
        pltpu.sync_copy(r0, y_hbm.at[pl.ds(base, TOK_W)])

    return sc_combine


def _sc_combine(pos0, pos1, wy):
    return _make_sc_combine()(pos0, pos1, wy)


def kernel(x, Wg, W1, W2):
    b, t, d = x.shape
    h = x.reshape(t, d)

    tw, d01, eotv = pl.pallas_call(
        _route_body,
        out_shape=(
            jax.ShapeDtypeStruct((P_MAX, E), jnp.float32),
            jax.ShapeDtypeStruct((TOKENS, E), jnp.int32),
            jax.ShapeDtypeStruct((G_MAX, E), jnp.int32),
        ),
    )(h, Wg)

    sidx = jnp.clip(tw[:, 0].astype(jnp.int32), 0, TOKENS - 1)
    eot = eotv[:, 0]
    valid = eotv[:, 1]
    pos0 = d01[:, 0]
    pos1 = d01[:, 1]

    xg = _sc_gather(sidx, h)

    wy = pl.pallas_call(
        _ffn_body,
        grid_spec=pltpu.PrefetchScalarGridSpec(
            num_scalar_prefetch=2,
            grid=(FF // FF_BLK, G_MAX),
            in_specs=[
                pl.BlockSpec((P_MAX, HIDDEN), lambda f, g, eot, val: (0, 0)),
                pl.BlockSpec((1, HIDDEN, FF_BLK),
                             lambda f, g, eot, val: (eot[g], 0, f)),
                pl.BlockSpec((1, FF_BLK, HIDDEN),
                             lambda f, g, eot, val: (eot[g], f, 0)),
                pl.BlockSpec((P_MAX, E), lambda f, g, eot, val: (0, 0)),
            ],
            out_specs=pl.BlockSpec((P_MAX, HIDDEN),
                                   lambda f, g, eot, val: (0, 0)),
        ),
        out_shape=jax.ShapeDtypeStruct((P_MAX, HIDDEN), jnp.float32),
    )(eot, valid, xg, W1, W2, tw)

    y = _sc_combine(pos0, pos1, wy)
    return y.reshape(b, t, d)


# speedup vs baseline: 1.3136x; 1.3136x over previous
import functools

import jax
import jax.numpy as jnp
from jax import lax
from jax.experimental import pallas as pl
from jax.experimental.pallas import tpu as pltpu
from jax.experimental.pallas import tpu_sc as plsc

HIDDEN = 768
FF = 3072
E = 8
TOKENS = 2048

T = 256
G_MAX = 24
P_MAX = G_MAX * T
PT = 512
FF_BLK = 512
NC = 2
NW = 32
ROWS_W = P_MAX // NW
GCH = 64
TOK_W = TOKENS // NW


def _route_body(h_ref, wg_ref, tw_ref, d01_ref, eotv_ref):
    h = h_ref[...]
    logits = jnp.dot(h, wg_ref[...], preferred_element_type=jnp.float32)
    ids = lax.broadcasted_iota(jnp.int32, logits.shape, 1)
    m0 = jnp.max(logits, axis=-1, keepdims=True)
    a0 = jnp.min(jnp.where(logits == m0, ids, E), axis=-1, keepdims=True)
    l2 = jnp.where(ids == a0, -jnp.inf, logits)
    m1 = jnp.max(l2, axis=-1, keepdims=True)
    a1 = jnp.min(jnp.where(l2 == m1, ids, E), axis=-1, keepdims=True)
    p1 = jnp.exp(m1 - m0)
    denom = 1.0 + p1
    w0 = 1.0 / denom
    w1 = p1 / denom

    oh0 = ids == a0
    oh1 = ids == a1
    s = oh0.astype(jnp.float32) + oh1.astype(jnp.float32)
    incl = s
    k = 1
    while k < TOKENS:
        shifted = jnp.concatenate(
            [jnp.zeros((k, E), jnp.float32), incl[: TOKENS - k, :]], axis=0)
        incl = incl + shifted
        k *= 2
    base = incl - s
    rank0 = jnp.sum(jnp.where(oh0, base, 0.0), axis=1, keepdims=True)
    rank1 = jnp.sum(jnp.where(oh1, base, 0.0), axis=1, keepdims=True)

    totals = incl[TOKENS - 1:TOKENS, :]
    totals_i = totals.astype(jnp.int32)
    tiles_i = (totals_i + (T - 1)) // T
    r8 = lax.broadcasted_iota(jnp.int32, (E, E), 0)
    c8 = lax.broadcasted_iota(jnp.int32, (E, E), 1)
    lt = (r8 <= c8).astype(jnp.float32)
    cumt = jnp.dot(tiles_i.astype(jnp.float32), lt,
                   preferred_element_type=jnp.float32)
    pbase = (cumt - tiles_i.astype(jnp.float32)) * float(T)

    dest0 = jnp.sum(jnp.where(oh0, pbase, 0.0), axis=1, keepdims=True) + rank0
    dest1 = jnp.sum(jnp.where(oh1, pbase, 0.0), axis=1, keepdims=True) + rank1
    dest0_i = dest0.astype(jnp.int32)
    dest1_i = dest1.astype(jnp.int32)
    d01_ref[...] = (jnp.where(ids == 0, dest0_i, 0)
                    + jnp.where(ids == 1, dest1_i, 0))

    cumt_i = cumt.astype(jnp.int32)
    g_col = lax.broadcasted_iota(jnp.int32, (G_MAX, 1), 0)
    eot_full = jnp.sum((cumt_i <= g_col).astype(jnp.int32),
                       axis=1, keepdims=True)
    total_tiles = cumt_i[:, E - 1:E]
    valid = (g_col < total_tiles).astype(jnp.int32)
    eotc = jnp.minimum(eot_full, E - 1)
    lane_g = lax.broadcasted_iota(jnp.int32, (G_MAX, E), 1)
    eotv_ref[...] = (jnp.where(lane_g == 0, eotc, 0)
                     + jnp.where(lane_g == 1, valid, 0))

    tid_f = lax.broadcasted_iota(jnp.int32, (TOKENS, 1), 0).astype(jnp.float32)
    v0 = jnp.where(ids == 0, tid_f, 0.0) + jnp.where(ids == 1, w0, 0.0)
    v1 = jnp.where(ids == 0, tid_f, 0.0) + jnp.where(ids == 1, w1, 0.0)
    dn = (((0,), (0,)), ((), ()))
    for i in range(P_MAX // PT):
        q = lax.broadcasted_iota(jnp.int32, (TOKENS, PT), 1) + i * PT
        m0f = (q == dest0_i).astype(jnp.float32)
        m1f = (q == dest1_i).astype(jnp.float32)
        tw_ref[i * PT:(i + 1) * PT, :] = (
            lax.dot_general(m0f, v0, dn, precision=lax.Precision.HIGHEST,
                            preferred_element_type=jnp.float32)
            + lax.dot_general(m1f, v1, dn, precision=lax.Precision.HIGHEST,
                              preferred_element_type=jnp.float32))


def _ffn_body(eot_ref, valid_ref, h_ref, w1_ref, w2_ref, tw_ref, y_ref,
              xs_ref):
    f = pl.program_id(0)
    g = pl.program_id(1)
    rows = pl.ds(g * T, T)

    @pl.when(valid_ref[g] == 1)
    def _():
        @pl.when(f == 0)
        def _():
            tw = tw_ref[rows, :]
            lane = lax.broadcasted_iota(jnp.int32, tw.shape, 1)
            tid = jnp.sum(jnp.where(lane == 0, tw, 0.0), axis=1,
                          keepdims=True).astype(jnp.int32)
            lane_t = lax.broadcasted_iota(jnp.int32, (T, TOKENS), 1)
            mb = (lane_t == tid).astype(jnp.bfloat16)
            xs_ref[rows, :] = jnp.dot(
                mb, h_ref[...],
                preferred_element_type=jnp.float32).astype(jnp.bfloat16)

        x = xs_ref[rows, :]
        w1 = w1_ref[0].astype(jnp.bfloat16)
        w2 = w2_ref[0].astype(jnp.bfloat16)
        pre = jnp.dot(x, w1, preferred_element_type=jnp.float32)
        act = (pre * jax.nn.sigmoid(pre)).astype(jnp.bfloat16)
        contrib = jnp.dot(act, w2, preferred_element_type=jnp.float32)

        @pl.when(f == 0)
        def _():
            y_ref[rows, :] = contrib

        @pl.when(jnp.logical_and(f > 0, f < FF // FF_BLK - 1))
        def _():
            y_ref[rows, :] += contrib

        @pl.when(f == FF // FF_BLK - 1)
        def _():
            tw = tw_ref[rows, :]
            lane = lax.broadcasted_iota(jnp.int32, tw.shape, 1)
            wt = jnp.sum(jnp.where(lane == 1, tw, 0.0), axis=1, keepdims=True)
            y_ref[rows, :] = (y_ref[rows, :] + contrib) * wt


@functools.lru_cache(maxsize=None)
def _make_sc_combine():
    mesh = plsc.VectorSubcoreMesh(core_axis_name="c", subcore_axis_name="s")

    @functools.partial(
        pl.kernel,
        out_type=jax.ShapeDtypeStruct((TOKENS, HIDDEN), jnp.float32),
        mesh=mesh,
        scratch_types=[
            pltpu.VMEM((TOK_W,), jnp.int32),
            pltpu.VMEM((TOK_W,), jnp.int32),
            pltpu.VMEM((TOK_W, HIDDEN), jnp.float32),
            pltpu.VMEM((TOK_W, HIDDEN), jnp.float32),
            pltpu.SemaphoreType.DMA,
            pltpu.SemaphoreType.DMA,
        ],
    )
    def sc_combine(pos0_hbm, pos1_hbm, rows_hbm, y_hbm,
                   i0, i1, r0, r1, s0, s1):
        wid = lax.axis_index("s") * NC + lax.axis_index("c")
        base = wid * TOK_W
        pltpu.sync_copy(pos0_hbm.at[pl.ds(base, TOK_W)], i0)
        pltpu.sync_copy(pos1_hbm.at[pl.ds(base, TOK_W)], i1)
        cp0 = pltpu.async_copy(rows_hbm.at[i0], r0, s0)
        cp1 = pltpu.async_copy(rows_hbm.at[i1], r1, s1)
        cp0.wait()
        cp1.wait()

        def add_row(t, carry):
            for j in range(HIDDEN // 16):
                sl = pl.ds(j * 16, 16)
                r0[t, sl] = r0[t, sl] + r1[t, sl]
            return carry

        lax.fori_loop(0, TOK_W, add_row, 0)
        pltpu.sync_copy(r0, y_hbm.at[pl.ds(base, TOK_W)])

    return sc_combine


def _sc_combine(pos0, pos1, wy):
    return _make_sc_combine()(pos0, pos1, wy)


def kernel(x, Wg, W1, W2):
    b, t, d = x.shape
    h = x.reshape(t, d)

    tw, d01, eotv = pl.pallas_call(
        _route_body,
        out_shape=(
            jax.ShapeDtypeStruct((P_MAX, E), jnp.float32),
            jax.ShapeDtypeStruct((TOKENS, E), jnp.int32),
            jax.ShapeDtypeStruct((G_MAX, E), jnp.int32),
        ),
    )(h, Wg)

    eot = eotv[:, 0]
    valid = eotv[:, 1]
    pos0 = d01[:, 0]
    pos1 = d01[:, 1]

    wy = pl.pallas_call(
        _ffn_body,
        grid_spec=pltpu.PrefetchScalarGridSpec(
            num_scalar_prefetch=2,
            grid=(FF // FF_BLK, G_MAX),
            in_specs=[
                pl.BlockSpec((TOKENS, HIDDEN), lambda f, g, eot, val: (0, 0)),
                pl.BlockSpec((1, HIDDEN, FF_BLK),
                             lambda f, g, eot, val: (eot[g], 0, f)),
                pl.BlockSpec((1, FF_BLK, HIDDEN),
                             lambda f, g, eot, val: (eot[g], f, 0)),
                pl.BlockSpec((P_MAX, E), lambda f, g, eot, val: (0, 0)),
            ],
            out_specs=pl.BlockSpec((P_MAX, HIDDEN),
                                   lambda f, g, eot, val: (0, 0)),
            scratch_shapes=[pltpu.VMEM((P_MAX, HIDDEN), jnp.bfloat16)],
        ),
        out_shape=jax.ShapeDtypeStruct((P_MAX, HIDDEN), jnp.float32),
    )(eot, valid, h.astype(jnp.bfloat16), W1, W2, tw)

    y = _sc_combine(pos0, pos1, wy)
    return y.reshape(b, t, d)


# speedup vs baseline: 1.7333x; 1.3195x over previous
import functools

import jax
import jax.numpy as jnp
from jax import lax
from jax.experimental import pallas as pl
from jax.experimental.pallas import tpu as pltpu
from jax.experimental.pallas import tpu_sc as plsc

HIDDEN = 768
FF = 3072
E = 8
TOKENS = 2048

T = 256
G_MAX = 24
P_MAX = G_MAX * T
PT = 512
FF_BLK = 512
NC = 2
NW = 32
ROWS_W = P_MAX // NW
GCH = 64
TOK_W = TOKENS // NW


def _route_body(h_ref, wg_ref, w01_ref, d01_ref, eotv_ref):
    h = h_ref[...]
    logits = jnp.dot(h, wg_ref[...], preferred_element_type=jnp.float32)
    ids = lax.broadcasted_iota(jnp.int32, logits.shape, 1)
    m0 = jnp.max(logits, axis=-1, keepdims=True)
    a0 = jnp.min(jnp.where(logits == m0, ids, E), axis=-1, keepdims=True)
    l2 = jnp.where(ids == a0, -jnp.inf, logits)
    m1 = jnp.max(l2, axis=-1, keepdims=True)
    a1 = jnp.min(jnp.where(l2 == m1, ids, E), axis=-1, keepdims=True)
    p1 = jnp.exp(m1 - m0)
    denom = 1.0 + p1
    w0 = 1.0 / denom
    w1 = p1 / denom

    oh0 = ids == a0
    oh1 = ids == a1
    s = oh0.astype(jnp.float32) + oh1.astype(jnp.float32)
    incl = s
    k = 1
    while k < TOKENS:
        shifted = jnp.concatenate(
            [jnp.zeros((k, E), jnp.float32), incl[: TOKENS - k, :]], axis=0)
        incl = incl + shifted
        k *= 2
    base = incl - s
    rank0 = jnp.sum(jnp.where(oh0, base, 0.0), axis=1, keepdims=True)
    rank1 = jnp.sum(jnp.where(oh1, base, 0.0), axis=1, keepdims=True)

    totals = incl[TOKENS - 1:TOKENS, :]
    totals_i = totals.astype(jnp.int32)
    tiles_i = (totals_i + (T - 1)) // T
    r8 = lax.broadcasted_iota(jnp.int32, (E, E), 0)
    c8 = lax.broadcasted_iota(jnp.int32, (E, E), 1)
    lt = (r8 <= c8).astype(jnp.float32)
    cumt = jnp.dot(tiles_i.astype(jnp.float32), lt,
                   preferred_element_type=jnp.float32)
    pbase = (cumt - tiles_i.astype(jnp.float32)) * float(T)

    dest0 = jnp.sum(jnp.where(oh0, pbase, 0.0), axis=1, keepdims=True) + rank0
    dest1 = jnp.sum(jnp.where(oh1, pbase, 0.0), axis=1, keepdims=True) + rank1
    dest0_i = dest0.astype(jnp.int32)
    dest1_i = dest1.astype(jnp.int32)
    d01_ref[...] = (jnp.where(ids == 0, dest0_i, 0)
                    + jnp.where(ids == 1, dest1_i, 0))

    cumt_i = cumt.astype(jnp.int32)
    g_col = lax.broadcasted_iota(jnp.int32, (G_MAX, 1), 0)
    eot_full = jnp.sum((cumt_i <= g_col).astype(jnp.int32),
                       axis=1, keepdims=True)
    total_tiles = cumt_i[:, E - 1:E]
    valid = (g_col < total_tiles).astype(jnp.int32)
    eotc = jnp.minimum(eot_full, E - 1)
    lane_g = lax.broadcasted_iota(jnp.int32, (G_MAX, E), 1)
    eotv_ref[...] = (jnp.where(lane_g == 0, eotc, 0)
                     + jnp.where(lane_g == 1, valid, 0))

    w01_ref[...] = jnp.where(ids == 0, w0, 0.0) + jnp.where(ids == 1, w1, 0.0)


def _ffn_body(eot_ref, valid_ref, h_ref, d0_ref, d1_ref, w01_ref,
              w1_ref, w2_ref, y_ref, xs_ref, ws_ref):
    f = pl.program_id(0)
    g = pl.program_id(1)
    rows = pl.ds(g * T, T)

    @pl.when(valid_ref[g] == 1)
    def _():
        @pl.when(f == 0)
        def _():
            q = lax.broadcasted_iota(jnp.int32, (T, 1), 0) + g * T
            m0 = (q == d0_ref[...]).astype(jnp.bfloat16)
            m1 = (q == d1_ref[...]).astype(jnp.bfloat16)
            xs_ref[rows, :] = jnp.dot(
                m0 + m1, h_ref[...],
                preferred_element_type=jnp.float32).astype(jnp.bfloat16)
            w01 = w01_ref[...].astype(jnp.bfloat16)
            wa = jnp.dot(m0, w01, preferred_element_type=jnp.float32)
            wb = jnp.dot(m1, w01, preferred_element_type=jnp.float32)
            lane = lax.broadcasted_iota(jnp.int32, wa.shape, 1)
            ws_ref[rows, :] = (jnp.where(lane == 0, wa, 0.0)
                               + jnp.where(lane == 1, wb, 0.0))

        x = xs_ref[rows, :]
        w1 = w1_ref[0].astype(jnp.bfloat16)
        w2 = w2_ref[0].astype(jnp.bfloat16)
        pre = jnp.dot(x, w1, preferred_element_type=jnp.float32)
        act = (pre * jax.nn.sigmoid(pre)).astype(jnp.bfloat16)
        contrib = jnp.dot(act, w2, preferred_element_type=jnp.float32)

        @pl.when(f == 0)
        def _():
            y_ref[rows, :] = contrib

        @pl.when(jnp.logical_and(f > 0, f < FF // FF_BLK - 1))
        def _():
            y_ref[rows, :] += contrib

        @pl.when(f == FF // FF_BLK - 1)
        def _():
            ws = ws_ref[rows, :]
            lane = lax.broadcasted_iota(jnp.int32, ws.shape, 1)
            wt = jnp.sum(jnp.where(lane < 2, ws, 0.0), axis=1, keepdims=True)
            y_ref[rows, :] = (y_ref[rows, :] + contrib) * wt


@functools.lru_cache(maxsize=None)
def _make_sc_combine():
    mesh = plsc.VectorSubcoreMesh(core_axis_name="c", subcore_axis_name="s")

    @functools.partial(
        pl.kernel,
        out_type=jax.ShapeDtypeStruct((TOKENS, HIDDEN), jnp.float32),
        mesh=mesh,
        scratch_types=[
            pltpu.VMEM((TOK_W,), jnp.int32),
            pltpu.VMEM((TOK_W,), jnp.int32),
            pltpu.VMEM((TOK_W, HIDDEN), jnp.float32),
            pltpu.VMEM((TOK_W, HIDDEN), jnp.float32),
            pltpu.SemaphoreType.DMA,
            pltpu.SemaphoreType.DMA,
        ],
    )
    def sc_combine(pos0_hbm, pos1_hbm, rows_hbm, y_hbm,
                   i0, i1, r0, r1, s0, s1):
        wid = lax.axis_index("s") * NC + lax.axis_index("c")
        base = wid * TOK_W
        pltpu.sync_copy(pos0_hbm.at[pl.ds(base, TOK_W)], i0)
        pltpu.sync_copy(pos1_hbm.at[pl.ds(base, TOK_W)], i1)
        cp0 = pltpu.async_copy(rows_hbm.at[i0], r0, s0)
        cp1 = pltpu.async_copy(rows_hbm.at[i1], r1, s1)
        cp0.wait()
        cp1.wait()

        def add_row(t, carry):
            for j in range(HIDDEN // 16):
                sl = pl.ds(j * 16, 16)
                r0[t, sl] = r0[t, sl] + r1[t, sl]
            return carry

        lax.fori_loop(0, TOK_W, add_row, 0)
        pltpu.sync_copy(r0, y_hbm.at[pl.ds(base, TOK_W)])

    return sc_combine


def _sc_combine(pos0, pos1, wy):
    return _make_sc_combine()(pos0, pos1, wy)


def kernel(x, Wg, W1, W2):
    b, t, d = x.shape
    h = x.reshape(t, d)

    w01, d01, eotv = pl.pallas_call(
        _route_body,
        out_shape=(
            jax.ShapeDtypeStruct((TOKENS, E), jnp.float32),
            jax.ShapeDtypeStruct((TOKENS, E), jnp.int32),
            jax.ShapeDtypeStruct((G_MAX, E), jnp.int32),
        ),
    )(h, Wg)

    eot = eotv[:, 0]
    valid = eotv[:, 1]
    pos0 = d01[:, 0]
    pos1 = d01[:, 1]
    d0_row = pos0.reshape(1, TOKENS)
    d1_row = pos1.reshape(1, TOKENS)

    wy = pl.pallas_call(
        _ffn_body,
        grid_spec=pltpu.PrefetchScalarGridSpec(
            num_scalar_prefetch=2,
            grid=(FF // FF_BLK, G_MAX),
            in_specs=[
                pl.BlockSpec((TOKENS, HIDDEN), lambda f, g, eot, val: (0, 0)),
                pl.BlockSpec((1, TOKENS), lambda f, g, eot, val: (0, 0)),
                pl.BlockSpec((1, TOKENS), lambda f, g, eot, val: (0, 0)),
                pl.BlockSpec((TOKENS, E), lambda f, g, eot, val: (0, 0)),
                pl.BlockSpec((1, HIDDEN, FF_BLK),
                             lambda f, g, eot, val: (eot[g], 0, f)),
                pl.BlockSpec((1, FF_BLK, HIDDEN),
                             lambda f, g, eot, val: (eot[g], f, 0)),
            ],
            out_specs=pl.BlockSpec((P_MAX, HIDDEN),
                                   lambda f, g, eot, val: (0, 0)),
            scratch_shapes=[
                pltpu.VMEM((P_MAX, HIDDEN), jnp.bfloat16),
                pltpu.VMEM((P_MAX, E), jnp.float32),
            ],
        ),
        out_shape=jax.ShapeDtypeStruct((P_MAX, HIDDEN), jnp.float32),
    )(eot, valid, h.astype(jnp.bfloat16), d0_row, d1_row, w01, W1, W2)

    y = _sc_combine(pos0, pos1, wy)
    return y.reshape(b, t, d)


# speedup vs baseline: 2.4524x; 1.4149x over previous
import functools

import jax
import jax.numpy as jnp
from jax import lax
from jax.experimental import pallas as pl
from jax.experimental.pallas import tpu as pltpu
from jax.experimental.pallas import tpu_sc as plsc

HIDDEN = 768
FF = 3072
E = 8
TOKENS = 2048

T = 256
G_MAX = 24
P_MAX = G_MAX * T
PT = 512
FF_BLK = 512
NC = 2
NW = 32
ROWS_W = P_MAX // NW
GCH = 64
TOK_W = TOKENS // NW


def _route_body(h_ref, wg_ref, w01_ref, d01_ref, eotv_ref):
    h = h_ref[...]
    logits = jnp.dot(h, wg_ref[...], preferred_element_type=jnp.float32)
    ids = lax.broadcasted_iota(jnp.int32, logits.shape, 1)
    m0 = jnp.max(logits, axis=-1, keepdims=True)
    a0 = jnp.min(jnp.where(logits == m0, ids, E), axis=-1, keepdims=True)
    l2 = jnp.where(ids == a0, -jnp.inf, logits)
    m1 = jnp.max(l2, axis=-1, keepdims=True)
    a1 = jnp.min(jnp.where(l2 == m1, ids, E), axis=-1, keepdims=True)
    p1 = jnp.exp(m1 - m0)
    denom = 1.0 + p1
    w0 = 1.0 / denom
    w1 = p1 / denom

    oh0 = ids == a0
    oh1 = ids == a1
    s = oh0.astype(jnp.float32) + oh1.astype(jnp.float32)
    incl = s
    k = 1
    while k < TOKENS:
        shifted = jnp.concatenate(
            [jnp.zeros((k, E), jnp.float32), incl[: TOKENS - k, :]], axis=0)
        incl = incl + shifted
        k *= 2
    base = incl - s
    rank0 = jnp.sum(jnp.where(oh0, base, 0.0), axis=1, keepdims=True)
    rank1 = jnp.sum(jnp.where(oh1, base, 0.0), axis=1, keepdims=True)

    totals = incl[TOKENS - 1:TOKENS, :]
    totals_i = totals.astype(jnp.int32)
    tiles_i = (totals_i + (T - 1)) // T
    r8 = lax.broadcasted_iota(jnp.int32, (E, E), 0)
    c8 = lax.broadcasted_iota(jnp.int32, (E, E), 1)
    lt = (r8 <= c8).astype(jnp.float32)
    cumt = jnp.dot(tiles_i.astype(jnp.float32), lt,
                   preferred_element_type=jnp.float32)
    pbase = (cumt - tiles_i.astype(jnp.float32)) * float(T)

    dest0 = jnp.sum(jnp.where(oh0, pbase, 0.0), axis=1, keepdims=True) + rank0
    dest1 = jnp.sum(jnp.where(oh1, pbase, 0.0), axis=1, keepdims=True) + rank1
    dest0_i = dest0.astype(jnp.int32)
    dest1_i = dest1.astype(jnp.int32)
    d01_ref[...] = (jnp.where(ids == 0, dest0_i, 0)
                    + jnp.where(ids == 1, dest1_i, 0))

    cumt_i = cumt.astype(jnp.int32)
    g_col = lax.broadcasted_iota(jnp.int32, (G_MAX, 1), 0)
    eot_full = jnp.sum((cumt_i <= g_col).astype(jnp.int32),
                       axis=1, keepdims=True)
    total_tiles = cumt_i[:, E - 1:E]
    valid = (g_col < total_tiles).astype(jnp.int32)
    eotc = jnp.minimum(eot_full, E - 1)
    lane_g = lax.broadcasted_iota(jnp.int32, (G_MAX, E), 1)
    eotv_ref[...] = (jnp.where(lane_g == 0, eotc, 0)
                     + jnp.where(lane_g == 1, valid, 0))

    w01_ref[...] = jnp.where(ids == 0, w0, 0.0) + jnp.where(ids == 1, w1, 0.0)


def _ffn_body(eot_ref, valid_ref, h_ref, d0_ref, d1_ref, w01_ref,
              w1_ref, w2_ref, y_ref):
    g = pl.program_id(0)

    @pl.when(valid_ref[g] == 1)
    def _():
        q = lax.broadcasted_iota(jnp.int32, (T, 1), 0) + g * T
        m0 = (q == d0_ref[...]).astype(jnp.bfloat16)
        m1 = (q == d1_ref[...]).astype(jnp.bfloat16)
        x = jnp.dot(m0 + m1, h_ref[...],
                    preferred_element_type=jnp.float32).astype(jnp.bfloat16)
        w01 = w01_ref[...].astype(jnp.bfloat16)
        wa = jnp.dot(m0, w01, preferred_element_type=jnp.float32)
        wb = jnp.dot(m1, w01, preferred_element_type=jnp.float32)
        lane = lax.broadcasted_iota(jnp.int32, wa.shape, 1)
        wt = jnp.sum(jnp.where(lane == 0, wa, 0.0)
                     + jnp.where(lane == 1, wb, 0.0), axis=1, keepdims=True)

        for fb in range(FF // FF_BLK):
            w1 = w1_ref[0, :, fb * FF_BLK:(fb + 1) * FF_BLK].astype(
                jnp.bfloat16)
            w2 = w2_ref[0, fb * FF_BLK:(fb + 1) * FF_BLK, :].astype(
                jnp.bfloat16)
            pre = jnp.dot(x, w1, preferred_element_type=jnp.float32)
            act = (pre * jax.nn.sigmoid(pre)).astype(jnp.bfloat16)
            contrib = jnp.dot(act, w2, preferred_element_type=jnp.float32)
            if fb == 0:
                acc = contrib
            else:
                acc = acc + contrib
        y_ref[...] = acc * wt


@functools.lru_cache(maxsize=None)
def _make_sc_combine():
    mesh = plsc.VectorSubcoreMesh(core_axis_name="c", subcore_axis_name="s")

    @functools.partial(
        pl.kernel,
        out_type=jax.ShapeDtypeStruct((TOKENS, HIDDEN), jnp.float32),
        mesh=mesh,
        scratch_types=[
            pltpu.VMEM((TOK_W,), jnp.int32),
            pltpu.VMEM((TOK_W,), jnp.int32),
            pltpu.VMEM((TOK_W, HIDDEN), jnp.float32),
            pltpu.VMEM((TOK_W, HIDDEN), jnp.float32),
            pltpu.SemaphoreType.DMA,
            pltpu.SemaphoreType.DMA,
        ],
    )
    def sc_combine(pos0_hbm, pos1_hbm, rows_hbm, y_hbm,
                   i0, i1, r0, r1, s0, s1):
        wid = lax.axis_index("s") * NC + lax.axis_index("c")
        base = wid * TOK_W
        pltpu.sync_copy(pos0_hbm.at[pl.ds(base, TOK_W)], i0)
        pltpu.sync_copy(pos1_hbm.at[pl.ds(base, TOK_W)], i1)
        cp0 = pltpu.async_copy(rows_hbm.at[i0], r0, s0)
        cp1 = pltpu.async_copy(rows_hbm.at[i1], r1, s1)
        cp0.wait()
        cp1.wait()

        def add_row(t, carry):
            for j in range(HIDDEN // 16):
                sl = pl.ds(j * 16, 16)
                r0[t, sl] = r0[t, sl] + r1[t, sl]
            return carry

        lax.fori_loop(0, TOK_W, add_row, 0)
        pltpu.sync_copy(r0, y_hbm.at[pl.ds(base, TOK_W)])

    return sc_combine


def _sc_combine(pos0, pos1, wy):
    return _make_sc_combine()(pos0, pos1, wy)


def kernel(x, Wg, W1, W2):
    b, t, d = x.shape
    h = x.reshape(t, d)

    w01, d01, eotv = pl.pallas_call(
        _route_body,
        out_shape=(
            jax.ShapeDtypeStruct((TOKENS, E), jnp.float32),
            jax.ShapeDtypeStruct((TOKENS, E), jnp.int32),
            jax.ShapeDtypeStruct((G_MAX, E), jnp.int32),
        ),
    )(h, Wg)

    eot = eotv[:, 0]
    valid = eotv[:, 1]
    pos0 = d01[:, 0]
    pos1 = d01[:, 1]
    d0_row = pos0.reshape(1, TOKENS)
    d1_row = pos1.reshape(1, TOKENS)

    wy = pl.pallas_call(
        _ffn_body,
        grid_spec=pltpu.PrefetchScalarGridSpec(
            num_scalar_prefetch=2,
            grid=(G_MAX,),
            in_specs=[
                pl.BlockSpec((TOKENS, HIDDEN), lambda g, eot, val: (0, 0)),
                pl.BlockSpec((1, TOKENS), lambda g, eot, val: (0, 0)),
                pl.BlockSpec((1, TOKENS), lambda g, eot, val: (0, 0)),
                pl.BlockSpec((TOKENS, E), lambda g, eot, val: (0, 0)),
                pl.BlockSpec((1, HIDDEN, FF),
                             lambda g, eot, val: (eot[g], 0, 0)),
                pl.BlockSpec((1, FF, HIDDEN),
                             lambda g, eot, val: (eot[g], 0, 0)),
            ],
            out_specs=pl.BlockSpec((T, HIDDEN),
                                   lambda g, eot, val: (g, 0)),
        ),
        out_shape=jax.ShapeDtypeStruct((P_MAX, HIDDEN), jnp.float32),
    )(eot, valid, h.astype(jnp.bfloat16), d0_row, d1_row, w01, W1, W2)

    y = _sc_combine(pos0, pos1, wy)
    return y.reshape(b, t, d)


# speedup vs baseline: 2.5158x; 1.0258x over previous
import functools

import jax
import jax.numpy as jnp
from jax import lax
from jax.experimental import pallas as pl
from jax.experimental.pallas import tpu as pltpu
from jax.experimental.pallas import tpu_sc as plsc

HIDDEN = 768
FF = 3072
E = 8
TOKENS = 2048

T = 256
G_MAX = 24
P_MAX = G_MAX * T
PT = 512
FF_BLK = 512
NC = 2
NW = 32
ROWS_W = P_MAX // NW
GCH = 64
TOK_W = TOKENS // NW


def _route_body(h_ref, wg_ref, w01_ref, d01_ref, eotv_ref, hbf_ref):
    h = h_ref[...]
    hbf_ref[...] = h.astype(jnp.bfloat16)
    logits = jnp.dot(h, wg_ref[...], preferred_element_type=jnp.float32)
    ids = lax.broadcasted_iota(jnp.int32, logits.shape, 1)
    m0 = jnp.max(logits, axis=-1, keepdims=True)
    a0 = jnp.min(jnp.where(logits == m0, ids, E), axis=-1, keepdims=True)
    l2 = jnp.where(ids == a0, -jnp.inf, logits)
    m1 = jnp.max(l2, axis=-1, keepdims=True)
    a1 = jnp.min(jnp.where(l2 == m1, ids, E), axis=-1, keepdims=True)
    p1 = jnp.exp(m1 - m0)
    denom = 1.0 + p1
    w0 = 1.0 / denom
    w1 = p1 / denom

    oh0 = ids == a0
    oh1 = ids == a1
    s = oh0.astype(jnp.float32) + oh1.astype(jnp.float32)
    incl = s
    k = 1
    while k < TOKENS:
        shifted = jnp.concatenate(
            [jnp.zeros((k, E), jnp.float32), incl[: TOKENS - k, :]], axis=0)
        incl = incl + shifted
        k *= 2
    base = incl - s
    rank0 = jnp.sum(jnp.where(oh0, base, 0.0), axis=1, keepdims=True)
    rank1 = jnp.sum(jnp.where(oh1, base, 0.0), axis=1, keepdims=True)

    totals = incl[TOKENS - 1:TOKENS, :]
    totals_i = totals.astype(jnp.int32)
    tiles_i = (totals_i + (T - 1)) // T
    r8 = lax.broadcasted_iota(jnp.int32, (E, E), 0)
    c8 = lax.broadcasted_iota(jnp.int32, (E, E), 1)
    lt = (r8 <= c8).astype(jnp.float32)
    cumt = jnp.dot(tiles_i.astype(jnp.float32), lt,
                   preferred_element_type=jnp.float32)
    pbase = (cumt - tiles_i.astype(jnp.float32)) * float(T)

    dest0 = jnp.sum(jnp.where(oh0, pbase, 0.0), axis=1, keepdims=True) + rank0
    dest1 = jnp.sum(jnp.where(oh1, pbase, 0.0), axis=1, keepdims=True) + rank1
    dest0_i = dest0.astype(jnp.int32)
    dest1_i = dest1.astype(jnp.int32)
    d01_ref[...] = (jnp.where(ids == 0, dest0_i, 0)
                    + jnp.where(ids == 1, dest1_i, 0))

    cumt_i = cumt.astype(jnp.int32)
    g_col = lax.broadcasted_iota(jnp.int32, (G_MAX, 1), 0)
    eot_full = jnp.sum((cumt_i <= g_col).astype(jnp.int32),
                       axis=1, keepdims=True)
    total_tiles = cumt_i[:, E - 1:E]
    valid = (g_col < total_tiles).astype(jnp.int32)
    eotc = jnp.minimum(eot_full, E - 1)
    lane_g = lax.broadcasted_iota(jnp.int32, (G_MAX, E), 1)
    eotv_ref[...] = (jnp.where(lane_g == 0, eotc, 0)
                     + jnp.where(lane_g == 1, valid, 0))

    w01_ref[...] = jnp.where(ids == 0, w0, 0.0) + jnp.where(ids == 1, w1, 0.0)


def _ffn_body(eot_ref, valid_ref, h_ref, d0_ref, d1_ref, w01_ref,
              w1_ref, w2_ref, y_ref):
    g = pl.program_id(0)

    @pl.when(valid_ref[g] == 1)
    def _():
        q = lax.broadcasted_iota(jnp.int32, (T, 1), 0) + g * T
        m0 = (q == d0_ref[...]).astype(jnp.bfloat16)
        m1 = (q == d1_ref[...]).astype(jnp.bfloat16)
        x = jnp.dot(m0 + m1, h_ref[...],
                    preferred_element_type=jnp.float32).astype(jnp.bfloat16)
        w01 = w01_ref[...].astype(jnp.bfloat16)
        wa = jnp.dot(m0, w01, preferred_element_type=jnp.float32)
        wb = jnp.dot(m1, w01, preferred_element_type=jnp.float32)
        lane = lax.broadcasted_iota(jnp.int32, wa.shape, 1)
        wt = jnp.sum(jnp.where(lane == 0, wa, 0.0)
                     + jnp.where(lane == 1, wb, 0.0), axis=1, keepdims=True)

        for fb in range(FF // FF_BLK):
            w1 = w1_ref[0, :, fb * FF_BLK:(fb + 1) * FF_BLK].astype(
                jnp.bfloat16)
            w2 = w2_ref[0, fb * FF_BLK:(fb + 1) * FF_BLK, :].astype(
                jnp.bfloat16)
            pre = jnp.dot(x, w1, preferred_element_type=jnp.float32)
            act = (pre * jax.nn.sigmoid(pre)).astype(jnp.bfloat16)
            contrib = jnp.dot(act, w2, preferred_element_type=jnp.float32)
            if fb == 0:
                acc = contrib
            else:
                acc = acc + contrib
        y_ref[...] = acc * wt


@functools.lru_cache(maxsize=None)
def _make_sc_combine():
    mesh = plsc.VectorSubcoreMesh(core_axis_name="c", subcore_axis_name="s")

    @functools.partial(
        pl.kernel,
        out_type=jax.ShapeDtypeStruct((TOKENS, HIDDEN), jnp.float32),
        mesh=mesh,
        scratch_types=[
            pltpu.VMEM((TOK_W,), jnp.int32),
            pltpu.VMEM((TOK_W,), jnp.int32),
            pltpu.VMEM((TOK_W, HIDDEN), jnp.float32),
            pltpu.VMEM((TOK_W, HIDDEN), jnp.float32),
            pltpu.SemaphoreType.DMA,
            pltpu.SemaphoreType.DMA,
        ],
    )
    def sc_combine(pos0_hbm, pos1_hbm, rows_hbm, y_hbm,
                   i0, i1, r0, r1, s0, s1):
        wid = lax.axis_index("s") * NC + lax.axis_index("c")
        base = wid * TOK_W
        pltpu.sync_copy(pos0_hbm.at[pl.ds(base, TOK_W)], i0)
        pltpu.sync_copy(pos1_hbm.at[pl.ds(base, TOK_W)], i1)
        cp0 = pltpu.async_copy(rows_hbm.at[i0], r0, s0)
        cp1 = pltpu.async_copy(rows_hbm.at[i1], r1, s1)
        cp0.wait()
        cp1.wait()

        @plsc.parallel_loop(0, TOK_W, 1, unroll=2)
        def _(t):
            for j in range(HIDDEN // 16):
                sl = pl.ds(j * 16, 16)
                r0[t, sl] = r0[t, sl] + r1[t, sl]
        pltpu.sync_copy(r0, y_hbm.at[pl.ds(base, TOK_W)])

    return sc_combine


def _sc_combine(pos0, pos1, wy):
    return _make_sc_combine()(pos0, pos1, wy)


def kernel(x, Wg, W1, W2):
    b, t, d = x.shape
    h = x.reshape(t, d)

    w01, d01, eotv, hbf = pl.pallas_call(
        _route_body,
        out_shape=(
            jax.ShapeDtypeStruct((TOKENS, E), jnp.float32),
            jax.ShapeDtypeStruct((TOKENS, E), jnp.int32),
            jax.ShapeDtypeStruct((G_MAX, E), jnp.int32),
            jax.ShapeDtypeStruct((TOKENS, HIDDEN), jnp.bfloat16),
        ),
    )(h, Wg)

    eot = eotv[:, 0]
    valid = eotv[:, 1]
    pos0 = d01[:, 0]
    pos1 = d01[:, 1]
    d0_row = pos0.reshape(1, TOKENS)
    d1_row = pos1.reshape(1, TOKENS)

    wy = pl.pallas_call(
        _ffn_body,
        grid_spec=pltpu.PrefetchScalarGridSpec(
            num_scalar_prefetch=2,
            grid=(G_MAX,),
            in_specs=[
                pl.BlockSpec((TOKENS, HIDDEN), lambda g, eot, val: (0, 0)),
                pl.BlockSpec((1, TOKENS), lambda g, eot, val: (0, 0)),
                pl.BlockSpec((1, TOKENS), lambda g, eot, val: (0, 0)),
                pl.BlockSpec((TOKENS, E), lambda g, eot, val: (0, 0)),
                pl.BlockSpec((1, HIDDEN, FF),
                             lambda g, eot, val: (eot[g], 0, 0)),
                pl.BlockSpec((1, FF, HIDDEN),
                             lambda g, eot, val: (eot[g], 0, 0)),
            ],
            out_specs=pl.BlockSpec((T, HIDDEN),
                                   lambda g, eot, val: (g, 0)),
        ),
        out_shape=jax.ShapeDtypeStruct((P_MAX, HIDDEN), jnp.float32),
    )(eot, valid, hbf, d0_row, d1_row, w01, W1, W2)

    y = _sc_combine(pos0, pos1, wy)
    return y.reshape(b, t, d)


# speedup vs baseline: 2.5175x; 1.0007x over previous
import functools

import jax
import jax.numpy as jnp
from jax import lax
from jax.experimental import pallas as pl
from jax.experimental.pallas import tpu as pltpu
from jax.experimental.pallas import tpu_sc as plsc

HIDDEN = 768
FF = 3072
E = 8
TOKENS = 2048

T = 256
G_MAX = 24
P_MAX = G_MAX * T
FF_BLK = 512
NC = 2
NW = 32
TOK_W = TOKENS // NW


def _route_body(h_ref, wg_ref, w01_ref, d01_ref, eotv_ref, hbf_ref):
    h = h_ref[...]
    hbf_ref[...] = h.astype(jnp.bfloat16)
    logits = jnp.dot(h, wg_ref[...], preferred_element_type=jnp.float32)
    ids = lax.broadcasted_iota(jnp.int32, logits.shape, 1)
    m0 = jnp.max(logits, axis=-1, keepdims=True)
    a0 = jnp.min(jnp.where(logits == m0, ids, E), axis=-1, keepdims=True)
    l2 = jnp.where(ids == a0, -jnp.inf, logits)
    m1 = jnp.max(l2, axis=-1, keepdims=True)
    a1 = jnp.min(jnp.where(l2 == m1, ids, E), axis=-1, keepdims=True)
    p1 = jnp.exp(m1 - m0)
    denom = 1.0 + p1
    w0 = 1.0 / denom
    w1 = p1 / denom

    oh0 = ids == a0
    oh1 = ids == a1
    s = oh0.astype(jnp.float32) + oh1.astype(jnp.float32)
    incl = s
    k = 1
    while k < TOKENS:
        shifted = jnp.concatenate(
            [jnp.zeros((k, E), jnp.float32), incl[: TOKENS - k, :]], axis=0)
        incl = incl + shifted
        k *= 2
    base = incl - s
    rank0 = jnp.sum(jnp.where(oh0, base, 0.0), axis=1, keepdims=True)
    rank1 = jnp.sum(jnp.where(oh1, base, 0.0), axis=1, keepdims=True)

    totals = incl[TOKENS - 1:TOKENS, :]
    totals_i = totals.astype(jnp.int32)
    tiles_i = (totals_i + (T - 1)) // T
    r8 = lax.broadcasted_iota(jnp.int32, (E, E), 0)
    c8 = lax.broadcasted_iota(jnp.int32, (E, E), 1)
    lt = (r8 <= c8).astype(jnp.float32)
    cumt = jnp.dot(tiles_i.astype(jnp.float32), lt,
                   preferred_element_type=jnp.float32)
    pbase = (cumt - tiles_i.astype(jnp.float32)) * float(T)

    dest0 = jnp.sum(jnp.where(oh0, pbase, 0.0), axis=1, keepdims=True) + rank0
    dest1 = jnp.sum(jnp.where(oh1, pbase, 0.0), axis=1, keepdims=True) + rank1
    dest0_i = dest0.astype(jnp.int32)
    dest1_i = dest1.astype(jnp.int32)
    d01_ref[...] = (jnp.where(ids == 0, dest0_i, 0)
                    + jnp.where(ids == 1, dest1_i, 0))

    cumt_i = cumt.astype(jnp.int32)
    g_col = lax.broadcasted_iota(jnp.int32, (G_MAX, 1), 0)
    eot_full = jnp.sum((cumt_i <= g_col).astype(jnp.int32),
                       axis=1, keepdims=True)
    total_tiles = cumt_i[:, E - 1:E]
    valid = (g_col < total_tiles).astype(jnp.int32)
    eotc = jnp.minimum(eot_full, E - 1)
    lane_g = lax.broadcasted_iota(jnp.int32, (G_MAX, E), 1)
    eotv_ref[...] = (jnp.where(lane_g == 0, eotc, 0)
                     + jnp.where(lane_g == 1, valid, 0))

    w01_ref[...] = jnp.where(ids == 0, w0, 0.0) + jnp.where(ids == 1, w1, 0.0)


def _ffn_body(eot_ref, valid_ref, h_ref, d0_ref, d1_ref, w01_ref,
              w1_ref, w2_ref, y_ref):
    g = pl.program_id(0)

    @pl.when(valid_ref[g] == 1)
    def _():
        q = lax.broadcasted_iota(jnp.int32, (T, 1), 0) + g * T
        m0 = (q == d0_ref[...]).astype(jnp.bfloat16)
        m1 = (q == d1_ref[...]).astype(jnp.bfloat16)
        x = jnp.dot(m0 + m1, h_ref[...],
                    preferred_element_type=jnp.float32).astype(jnp.bfloat16)
        w01 = w01_ref[...].astype(jnp.bfloat16)
        wa = jnp.dot(m0, w01, preferred_element_type=jnp.float32)
        wb = jnp.dot(m1, w01, preferred_element_type=jnp.float32)
        lane = lax.broadcasted_iota(jnp.int32, wa.shape, 1)
        wt = jnp.sum(jnp.where(lane == 0, wa, 0.0)
                     + jnp.where(lane == 1, wb, 0.0), axis=1, keepdims=True)

        for fb in range(FF // FF_BLK):
            w1 = w1_ref[0, :, fb * FF_BLK:(fb + 1) * FF_BLK].astype(
                jnp.bfloat16)
            w2 = w2_ref[0, fb * FF_BLK:(fb + 1) * FF_BLK, :].astype(
                jnp.bfloat16)
            pre = jnp.dot(x, w1, preferred_element_type=jnp.float32)
            act = (pre * jax.nn.sigmoid(pre)).astype(jnp.bfloat16)
            contrib = jnp.dot(act, w2, preferred_element_type=jnp.float32)
            if fb == 0:
                acc = contrib
            else:
                acc = acc + contrib
        y_ref[...] = acc * wt


@functools.lru_cache(maxsize=None)
def _make_sc_combine():
    mesh = plsc.VectorSubcoreMesh(core_axis_name="c", subcore_axis_name="s")

    @functools.partial(
        pl.kernel,
        out_type=jax.ShapeDtypeStruct((TOKENS, HIDDEN), jnp.float32),
        mesh=mesh,
        scratch_types=[
            pltpu.VMEM((TOK_W,), jnp.int32),
            pltpu.VMEM((TOK_W,), jnp.int32),
            pltpu.VMEM((TOK_W, HIDDEN), jnp.float32),
            pltpu.VMEM((TOK_W, HIDDEN), jnp.float32),
            pltpu.SemaphoreType.DMA,
            pltpu.SemaphoreType.DMA,
        ],
    )
    def sc_combine(pos0_hbm, pos1_hbm, rows_hbm, y_hbm,
                   i0, i1, r0, r1, s0, s1):
        wid = lax.axis_index("s") * NC + lax.axis_index("c")
        base = wid * TOK_W
        pltpu.sync_copy(pos0_hbm.at[pl.ds(base, TOK_W)], i0)
        pltpu.sync_copy(pos1_hbm.at[pl.ds(base, TOK_W)], i1)
        cp0 = pltpu.async_copy(rows_hbm.at[i0], r0, s0)
        cp1 = pltpu.async_copy(rows_hbm.at[i1], r1, s1)
        cp0.wait()
        cp1.wait()

        @plsc.parallel_loop(0, TOK_W, 1, unroll=2)
        def _(t):
            for j in range(HIDDEN // 16):
                sl = pl.ds(j * 16, 16)
                r0[t, sl] = r0[t, sl] + r1[t, sl]
        pltpu.sync_copy(r0, y_hbm.at[pl.ds(base, TOK_W)])

    return sc_combine


def _sc_combine(pos0, pos1, wy):
    return _make_sc_combine()(pos0, pos1, wy)


def kernel(x, Wg, W1, W2):
    b, t, d = x.shape
    h = x.reshape(t, d)

    w01, d01, eotv, hbf = pl.pallas_call(
        _route_body,
        out_shape=(
            jax.ShapeDtypeStruct((TOKENS, E), jnp.float32),
            jax.ShapeDtypeStruct((TOKENS, E), jnp.int32),
            jax.ShapeDtypeStruct((G_MAX, E), jnp.int32),
            jax.ShapeDtypeStruct((TOKENS, HIDDEN), jnp.bfloat16),
        ),
    )(h, Wg)

    eot = eotv[:, 0]
    valid = eotv[:, 1]
    pos0 = d01[:, 0]
    pos1 = d01[:, 1]
    d0_row = pos0.reshape(1, TOKENS)
    d1_row = pos1.reshape(1, TOKENS)

    wy = pl.pallas_call(
        _ffn_body,
        grid_spec=pltpu.PrefetchScalarGridSpec(
            num_scalar_prefetch=2,
            grid=(G_MAX,),
            in_specs=[
                pl.BlockSpec((TOKENS, HIDDEN), lambda g, eot, val: (0, 0)),
                pl.BlockSpec((1, TOKENS), lambda g, eot, val: (0, 0)),
                pl.BlockSpec((1, TOKENS), lambda g, eot, val: (0, 0)),
                pl.BlockSpec((TOKENS, E), lambda g, eot, val: (0, 0)),
                pl.BlockSpec((1, HIDDEN, FF),
                             lambda g, eot, val: (eot[g], 0, 0)),
                pl.BlockSpec((1, FF, HIDDEN),
                             lambda g, eot, val: (eot[g], 0, 0)),
            ],
            out_specs=pl.BlockSpec((T, HIDDEN),
                                   lambda g, eot, val: (g, 0)),
        ),
        out_shape=jax.ShapeDtypeStruct((P_MAX, HIDDEN), jnp.float32),
    )(eot, valid, hbf, d0_row, d1_row, w01, W1, W2)

    y = _sc_combine(pos0, pos1, wy)
    return y.reshape(b, t, d)
